# baseline (speedup 1.0000x reference)
import functools

import jax
import jax.numpy as jnp
from jax import lax
from jax.experimental import pallas as pl
from jax.experimental.pallas import tpu as pltpu
from jax.experimental.pallas import tpu_sc as plsc

O_CH, I_CH, KS = 192, 192, 3
NW = O_CH * I_CH * KS * KS
NEMB = 256
WROWS = NW // 128
BW = 288
GVQ = WROWS // BW
SUB = 32
NSUB = BW // SUB
COMMIT = 0.25
MAXV = 255.0
SP = 30
OS = 28
B = 4


def _sort_body(cb_row_ref, cb_col_ref, s_ref, m_ref, d_ref):
    row = cb_row_ref[...]
    col = cb_col_ref[...]
    ii = jax.lax.broadcasted_iota(jnp.int32, (NEMB, NEMB), 0)
    jj = jax.lax.broadcasted_iota(jnp.int32, (NEMB, NEMB), 1)
    less = (row < col) | ((row == col) & (jj < ii))
    rank = jnp.sum(less.astype(jnp.int32), axis=1, keepdims=True)
    s = jnp.sum(jnp.where(rank == jj, col, 0.0), axis=0, keepdims=True)
    s_next = jnp.sum(jnp.where(rank == jj + 1, col, 0.0), axis=0, keepdims=True)
    lane = jax.lax.broadcasted_iota(jnp.int32, (1, NEMB), 1)
    inf = jnp.float32(jnp.inf)
    s_ref[...] = s
    m_ref[...] = jnp.where(lane == NEMB - 1, inf, (s + s_next) * 0.5)
    d_ref[...] = jnp.where(lane == NEMB - 1, 0.0, s_next - s)


NWORK = 32
NWPW = NW // NWORK
SUBV = 2
CHUNK = SUBV * 16

_GDN = lax.GatherDimensionNumbers(
    offset_dims=(), collapsed_slice_dims=(0,), start_index_map=(0,))


def _vperm(vec, idx):
    return lax.gather(vec, idx[:, None], dimension_numbers=_GDN,
                      slice_sizes=(1,),
                      mode=lax.GatherScatterMode.PROMISE_IN_BOUNDS)


NACT = 27
NROWPW = WROWS // NACT


def _sc_vq_body(w_hbm, m_hbm, s_hbm, q_hbm, red_hbm, w_v, q_v, m_v, s_v,
                red_v):
    wid = lax.axis_index("s") * 2 + lax.axis_index("c")
    base = wid * NROWPW
    pltpu.sync_copy(m_hbm, m_v)
    pltpu.sync_copy(s_hbm, s_v)
    lane = jax.lax.broadcasted_iota(jnp.int32, (16,), 0)
    zero = jnp.zeros((16,), jnp.float32)
    mrows = [m_v[b, :] for b in range(16)]
    mcol = []
    for r in range(16):
        col = zero
        ridx = jnp.full((16,), r, jnp.int32)
        for b in range(16):
            col = jnp.where(lane == b, _vperm(mrows[b], ridx), col)
        mcol.append(col)
    m127 = _vperm(mrows[7], jnp.full((16,), 15, jnp.int32))

    def search(w):
        pos = jnp.where(m127 < w, 128, 0)
        gm = _vperm(mcol[15], (pos >> 4) + 3)
        pos = jnp.where(gm < w, pos + 64, pos)
        gm = _vperm(mcol[15], (pos >> 4) + 1)
        pos = jnp.where(gm < w, pos + 32, pos)
        gm = _vperm(mcol[15], pos >> 4)
        pos = jnp.where(gm < w, pos + 16, pos)
        row = pos >> 4
        gm = _vperm(mcol[7], row)
        pos = jnp.where(gm < w, pos + 8, pos)
        gm = jnp.where((pos & 8) != 0, _vperm(mcol[11], row),
                       _vperm(mcol[3], row))
        pos = jnp.where(gm < w, pos + 4, pos)
        ga = jnp.where((pos & 4) != 0, _vperm(mcol[5], row),
                       _vperm(mcol[1], row))
        gb = jnp.where((pos & 4) != 0, _vperm(mcol[13], row),
                       _vperm(mcol[9], row))
        gm = jnp.where((pos & 8) != 0, gb, ga)
        pos = jnp.where(gm < w, pos + 2, pos)
        t0 = jnp.where((pos & 2) != 0, _vperm(mcol[2], row),
                       _vperm(mcol[0], row))
        t1 = jnp.where((pos & 2) != 0, _vperm(mcol[6], row),
                       _vperm(mcol[4], row))
        t2 = jnp.where((pos & 2) != 0, _vperm(mcol[10], row),
                       _vperm(mcol[8], row))
        t3 = jnp.where((pos & 2) != 0, _vperm(mcol[14], row),
                       _vperm(mcol[12], row))
        u0 = jnp.where((pos & 4) != 0, t1, t0)
        u1 = jnp.where((pos & 4) != 0, t3, t2)
        gm = jnp.where((pos & 8) != 0, u1, u0)
        pos = jnp.where(gm < w, pos + 1, pos)
        f = pos & 15
        q = zero
        for b in range(16):
            q = jnp.where(row == b, _vperm(s_v[b, :], f), q)
        return q

    red_v[0, :] = zero
    red_v[1, :] = zero
    red_v[2, :] = zero
    init = (zero,) * (3 * SUBV)

    def trip(r, acc):
        accs = list(acc)
        for j in range(8):
            w = w_v[r, pl.ds(j * 16, 16)]
            q = search(w)
            q_v[r, pl.ds(j * 16, 16)] = q
            e = q - w
            k = 3 * (j % SUBV)
            accs[k] = accs[k] + e * e
            accs[k + 1] = accs[k + 1] + (w * w + q * q - 2.0 * w * q)
            accs[k + 2] = jnp.maximum(accs[k + 2], jnp.abs(q))
        return tuple(accs)

    @pl.when(wid < NACT)
    def _():
        pltpu.sync_copy(w_hbm.at[pl.ds(base, NROWPW), :], w_v)
        acc = lax.fori_loop(0, NROWPW, trip, init)
        red_v[0, :] = acc[0] + acc[3]
        red_v[1, :] = acc[1] + acc[4]
        red_v[2, :] = jnp.maximum(acc[2], acc[5])
        pltpu.sync_copy(q_v, q_hbm.at[pl.ds(base, NROWPW), :])

    pltpu.sync_copy(red_v, red_hbm.at[wid])


def _conv_body(x_ref, w9_ref, red_ref, out_ref, loss_ref):
    b = pl.program_id(0)
    red = red_ref[...]
    esum = jnp.sum(red[:, 0, :])
    dsum = jnp.sum(red[:, 1, :])
    qmax = jnp.max(red[:, 2, :])
    max_abs = jnp.where(qmax > 0.0, qmax, 1.0)

    w9 = w9_ref[...]
    wpos = jnp.maximum(w9, 0.0)
    wneg = jnp.maximum(-w9, 0.0)
    wint = jnp.round(wpos / max_abs * MAXV) - jnp.round(wneg / max_abs * MAXV)
    wb = wint.astype(jnp.bfloat16)

    x = x_ref[0]
    xq = jnp.round(jnp.clip(x, -8.0, 8.0 - 0.0625) * 16.0)
    xb = xq.astype(jnp.bfloat16)

    acc = jnp.zeros((OS * OS, O_CH), jnp.float32)
    for t in range(KS * KS):
        dy, dx = t // KS, t % KS
        p = jax.lax.dot(xb, wb[t], preferred_element_type=jnp.float32)
        pw = p.reshape(SP, SP, O_CH)[dy:dy + OS, dx:dx + OS, :]
        acc = acc + pw.reshape(OS * OS, O_CH)
    out_ref[0] = acc * (max_abs / (MAXV * 16.0))

    @pl.when(b == 0)
    def _():
        e_l = esum / NW
        avg = dsum / NW
        scale = jnp.where(avg < 0.001, 0.1, jnp.where(avg < 0.01, 0.5, 1.0))
        loss = e_l + COMMIT * scale * e_l
        loss_ref[...] = jnp.full((1, 128), loss)


def kernel(x, weight, codebook):
    cb_row = codebook.reshape(1, NEMB)
    cb_col = codebook.reshape(NEMB, 1)
    s, m, d = pl.pallas_call(
        _sort_body,
        out_shape=[jax.ShapeDtypeStruct((1, NEMB), jnp.float32)] * 3,
    )(cb_row, cb_col)

    sc_vq = functools.partial(
        pl.kernel,
        out_type=[jax.ShapeDtypeStruct((WROWS, 128), jnp.float32),
                  jax.ShapeDtypeStruct((NWORK, 3, 16), jnp.float32)],
        mesh=plsc.VectorSubcoreMesh(core_axis_name="c", subcore_axis_name="s"),
        scratch_types=[pltpu.VMEM((96, 128), jnp.float32),
                       pltpu.VMEM((96, 128), jnp.float32),
                       pltpu.VMEM((16, 16), jnp.float32),
                       pltpu.VMEM((16, 16), jnp.float32),
                       pltpu.VMEM((3, 16), jnp.float32)],
    )(_sc_vq_body)
    q_flat, red = sc_vq(weight.reshape(WROWS, 128), m.reshape(16, 16),
                        s.reshape(16, 16))

    w9 = q_flat.reshape(O_CH, I_CH, KS * KS).transpose(2, 1, 0)
    xpad = jnp.pad(x, ((0, 0), (0, 0), (1, 1), (1, 1)))
    xt = xpad.transpose(0, 2, 3, 1).reshape(B, SP * SP, I_CH)

    out_t, loss_arr = pl.pallas_call(
        _conv_body,
        grid=(B,),
        in_specs=[pl.BlockSpec((1, SP * SP, I_CH), lambda b: (b, 0, 0)),
                  pl.BlockSpec((KS * KS, I_CH, O_CH), lambda b: (0, 0, 0)),
                  pl.BlockSpec((NWORK, 3, 16), lambda b: (0, 0, 0))],
        out_specs=[pl.BlockSpec((1, OS * OS, O_CH), lambda b: (b, 0, 0)),
                   pl.BlockSpec((1, 128), lambda b: (0, 0))],
        out_shape=[jax.ShapeDtypeStruct((B, OS * OS, O_CH), jnp.float32),
                   jax.ShapeDtypeStruct((1, 128), jnp.float32)],
    )(xt, w9, red)

    out = out_t.transpose(0, 2, 1).reshape(B, O_CH, OS, OS)
    return out, loss_arr[0, 0]


# speedup vs baseline: 2.7544x; 2.7544x over previous
import functools

import jax
import jax.numpy as jnp
from jax import lax
from jax.experimental import pallas as pl
from jax.experimental.pallas import tpu as pltpu
from jax.experimental.pallas import tpu_sc as plsc

O_CH, I_CH, KS = 192, 192, 3
NW = O_CH * I_CH * KS * KS
NEMB = 256
WROWS = NW // 128
BW = 288
GVQ = WROWS // BW
SUB = 32
NSUB = BW // SUB
COMMIT = 0.25
MAXV = 255.0
SP = 30
OS = 28
B = 4


def _sort_body(cb_row_ref, cb_col_ref, s_ref, m_ref, d_ref):
    row = cb_row_ref[...]
    col = cb_col_ref[...]
    ii = jax.lax.broadcasted_iota(jnp.int32, (NEMB, NEMB), 0)
    jj = jax.lax.broadcasted_iota(jnp.int32, (NEMB, NEMB), 1)
    less = (row < col) | ((row == col) & (jj < ii))
    rank = jnp.sum(less.astype(jnp.int32), axis=1, keepdims=True)
    s = jnp.sum(jnp.where(rank == jj, col, 0.0), axis=0, keepdims=True)
    s_next = jnp.sum(jnp.where(rank == jj + 1, col, 0.0), axis=0, keepdims=True)
    lane = jax.lax.broadcasted_iota(jnp.int32, (1, NEMB), 1)
    inf = jnp.float32(jnp.inf)
    s_ref[...] = s
    m_ref[...] = jnp.where(lane == NEMB - 1, inf, (s + s_next) * 0.5)
    d_ref[...] = jnp.where(lane == NEMB - 1, 0.0, s_next - s)


NWORK = 32
NWPW = NW // NWORK
SUBV = 2
CHUNK = SUBV * 16

_GDN = lax.GatherDimensionNumbers(
    offset_dims=(), collapsed_slice_dims=(0,), start_index_map=(0,))


def _vperm(vec, idx):
    return lax.gather(vec, idx[:, None], dimension_numbers=_GDN,
                      slice_sizes=(1,),
                      mode=lax.GatherScatterMode.PROMISE_IN_BOUNDS)


NACT = 27
WR2 = KS * KS * I_CH
NROWPW = WR2 // NACT
NVEC = O_CH // 16


def _sc_vq_body(w_hbm, m_hbm, s_hbm, q_hbm, red_hbm, w_v, q_v, m_v, s_v,
                red_v):
    wid = lax.axis_index("s") * 2 + lax.axis_index("c")
    base = wid * NROWPW
    pltpu.sync_copy(m_hbm, m_v)
    pltpu.sync_copy(s_hbm, s_v)
    lane = jax.lax.broadcasted_iota(jnp.int32, (16,), 0)
    zero = jnp.zeros((16,), jnp.float32)
    mrows = [m_v[b, :] for b in range(16)]
    mcol = []
    for r in range(16):
        col = zero
        ridx = jnp.full((16,), r, jnp.int32)
        for b in range(16):
            col = jnp.where(lane == b, _vperm(mrows[b], ridx), col)
        mcol.append(col)
    m127 = _vperm(mrows[7], jnp.full((16,), 15, jnp.int32))

    def search(w):
        pos = jnp.where(m127 < w, 128, 0)
        gm = _vperm(mcol[15], (pos >> 4) + 3)
        pos = jnp.where(gm < w, pos + 64, pos)
        gm = _vperm(mcol[15], (pos >> 4) + 1)
        pos = jnp.where(gm < w, pos + 32, pos)
        gm = _vperm(mcol[15], pos >> 4)
        pos = jnp.where(gm < w, pos + 16, pos)
        row = pos >> 4
        gm = _vperm(mcol[7], row)
        pos = jnp.where(gm < w, pos + 8, pos)
        gm = jnp.where((pos & 8) != 0, _vperm(mcol[11], row),
                       _vperm(mcol[3], row))
        pos = jnp.where(gm < w, pos + 4, pos)
        ga = jnp.where((pos & 4) != 0, _vperm(mcol[5], row),
                       _vperm(mcol[1], row))
        gb = jnp.where((pos & 4) != 0, _vperm(mcol[13], row),
                       _vperm(mcol[9], row))
        gm = jnp.where((pos & 8) != 0, gb, ga)
        pos = jnp.where(gm < w, pos + 2, pos)
        t0 = jnp.where((pos & 2) != 0, _vperm(mcol[2], row),
                       _vperm(mcol[0], row))
        t1 = jnp.where((pos & 2) != 0, _vperm(mcol[6], row),
                       _vperm(mcol[4], row))
        t2 = jnp.where((pos & 2) != 0, _vperm(mcol[10], row),
                       _vperm(mcol[8], row))
        t3 = jnp.where((pos & 2) != 0, _vperm(mcol[14], row),
                       _vperm(mcol[12], row))
        u0 = jnp.where((pos & 4) != 0, t1, t0)
        u1 = jnp.where((pos & 4) != 0, t3, t2)
        gm = jnp.where((pos & 8) != 0, u1, u0)
        pos = jnp.where(gm < w, pos + 1, pos)
        f = pos & 15
        q = zero
        for b in range(16):
            q = jnp.where(row == b, _vperm(s_v[b, :], f), q)
        return q

    red_v[0, :] = zero
    red_v[1, :] = zero
    red_v[2, :] = zero
    init = (zero,) * (3 * SUBV)

    def trip(r, acc):
        accs = list(acc)
        for j in range(NVEC):
            w = w_v[r, pl.ds(j * 16, 16)]
            q = search(w)
            q_v[r, pl.ds(j * 16, 16)] = q
            e = q - w
            k = 3 * (j % SUBV)
            accs[k] = accs[k] + e * e
            accs[k + 1] = accs[k + 1] + (w * w + q * q - 2.0 * w * q)
            accs[k + 2] = jnp.maximum(accs[k + 2], jnp.abs(q))
        return tuple(accs)

    @pl.when(wid < NACT)
    def _():
        pltpu.sync_copy(w_hbm.at[pl.ds(base, NROWPW), :], w_v)
        acc = lax.fori_loop(0, NROWPW, trip, init)
        red_v[0, :] = acc[0] + acc[3]
        red_v[1, :] = acc[1] + acc[4]
        red_v[2, :] = jnp.maximum(acc[2], acc[5])
        pltpu.sync_copy(q_v, q_hbm.at[pl.ds(base, NROWPW), :])

    pltpu.sync_copy(red_v, red_hbm.at[wid])


def _conv_body(x_ref, w9_ref, red_ref, out_ref, loss_ref):
    b = pl.program_id(0)
    red = red_ref[...]
    esum = jnp.sum(red[:, 0, :])
    dsum = jnp.sum(red[:, 1, :])
    qmax = jnp.max(red[:, 2, :])
    max_abs = jnp.where(qmax > 0.0, qmax, 1.0)

    w9 = w9_ref[...]
    wpos = jnp.maximum(w9, 0.0)
    wneg = jnp.maximum(-w9, 0.0)
    wint = jnp.round(wpos / max_abs * MAXV) - jnp.round(wneg / max_abs * MAXV)
    wb = wint.astype(jnp.bfloat16)

    x = x_ref[0]
    xq = jnp.round(jnp.clip(x, -8.0, 8.0 - 0.0625) * 16.0)
    xb = xq.astype(jnp.bfloat16)

    acc = jnp.zeros((OS * OS, O_CH), jnp.float32)
    for t in range(KS * KS):
        dy, dx = t // KS, t % KS
        p = jax.lax.dot(xb, wb[t], preferred_element_type=jnp.float32)
        pw = p.reshape(SP, SP, O_CH)[dy:dy + OS, dx:dx + OS, :]
        acc = acc + pw.reshape(OS * OS, O_CH)
    out_ref[0] = acc * (max_abs / (MAXV * 16.0))

    @pl.when(b == 0)
    def _():
        e_l = esum / NW
        avg = dsum / NW
        scale = jnp.where(avg < 0.001, 0.1, jnp.where(avg < 0.01, 0.5, 1.0))
        loss = e_l + COMMIT * scale * e_l
        loss_ref[...] = jnp.full((1, 128), loss)


def kernel(x, weight, codebook):
    cb_row = codebook.reshape(1, NEMB)
    cb_col = codebook.reshape(NEMB, 1)
    s, m, d = pl.pallas_call(
        _sort_body,
        out_shape=[jax.ShapeDtypeStruct((1, NEMB), jnp.float32)] * 3,
    )(cb_row, cb_col)

    sc_vq = functools.partial(
        pl.kernel,
        out_type=[jax.ShapeDtypeStruct((WR2, O_CH), jnp.float32),
                  jax.ShapeDtypeStruct((NWORK, 3, 16), jnp.float32)],
        mesh=plsc.VectorSubcoreMesh(core_axis_name="c", subcore_axis_name="s"),
        scratch_types=[pltpu.VMEM((NROWPW, O_CH), jnp.float32),
                       pltpu.VMEM((NROWPW, O_CH), jnp.float32),
                       pltpu.VMEM((16, 16), jnp.float32),
                       pltpu.VMEM((16, 16), jnp.float32),
                       pltpu.VMEM((3, 16), jnp.float32)],
    )(_sc_vq_body)
    w_tco = weight.reshape(O_CH, I_CH, KS * KS).transpose(2, 1, 0)
    q_flat, red = sc_vq(w_tco.reshape(WR2, O_CH), m.reshape(16, 16),
                        s.reshape(16, 16))

    w9 = q_flat.reshape(KS * KS, I_CH, O_CH)
    xpad = jnp.pad(x, ((0, 0), (0, 0), (1, 1), (1, 1)))
    xt = xpad.transpose(0, 2, 3, 1).reshape(B, SP * SP, I_CH)

    out_t, loss_arr = pl.pallas_call(
        _conv_body,
        grid=(B,),
        in_specs=[pl.BlockSpec((1, SP * SP, I_CH), lambda b: (b, 0, 0)),
                  pl.BlockSpec((KS * KS, I_CH, O_CH), lambda b: (0, 0, 0)),
                  pl.BlockSpec((NWORK, 3, 16), lambda b: (0, 0, 0))],
        out_specs=[pl.BlockSpec((1, OS * OS, O_CH), lambda b: (b, 0, 0)),
                   pl.BlockSpec((1, 128), lambda b: (0, 0))],
        out_shape=[jax.ShapeDtypeStruct((B, OS * OS, O_CH), jnp.float32),
                   jax.ShapeDtypeStruct((1, 128), jnp.float32)],
    )(xt, w9, red)

    out = out_t.transpose(0, 2, 1).reshape(B, O_CH, OS, OS)
    return out, loss_arr[0, 0]


# speedup vs baseline: 2.9876x; 1.0847x over previous
import functools

import jax
import jax.numpy as jnp
from jax import lax
from jax.experimental import pallas as pl
from jax.experimental.pallas import tpu as pltpu
from jax.experimental.pallas import tpu_sc as plsc

O_CH, I_CH, KS = 192, 192, 3
NW = O_CH * I_CH * KS * KS
NEMB = 256
WROWS = NW // 128
BW = 288
GVQ = WROWS // BW
SUB = 32
NSUB = BW // SUB
COMMIT = 0.25
MAXV = 255.0
SP = 30
OS = 28
B = 4


def _sort_body(cb_row_ref, cb_col_ref, s_ref, m_ref, d_ref):
    row = cb_row_ref[...]
    col = cb_col_ref[...]
    ii = jax.lax.broadcasted_iota(jnp.int32, (NEMB, NEMB), 0)
    jj = jax.lax.broadcasted_iota(jnp.int32, (NEMB, NEMB), 1)
    less = (row < col) | ((row == col) & (jj < ii))
    rank = jnp.sum(less.astype(jnp.int32), axis=1, keepdims=True)
    s = jnp.sum(jnp.where(rank == jj, col, 0.0), axis=0, keepdims=True)
    s_next = jnp.sum(jnp.where(rank == jj + 1, col, 0.0), axis=0, keepdims=True)
    lane = jax.lax.broadcasted_iota(jnp.int32, (1, NEMB), 1)
    inf = jnp.float32(jnp.inf)
    s_ref[...] = s
    m_ref[...] = jnp.where(lane == NEMB - 1, inf, (s + s_next) * 0.5)
    d_ref[...] = jnp.where(lane == NEMB - 1, 0.0, s_next - s)


NWORK = 32
NWPW = NW // NWORK
SUBV = 2
CHUNK = SUBV * 16

_GDN = lax.GatherDimensionNumbers(
    offset_dims=(), collapsed_slice_dims=(0,), start_index_map=(0,))


def _vperm(vec, idx):
    return lax.gather(vec, idx[:, None], dimension_numbers=_GDN,
                      slice_sizes=(1,),
                      mode=lax.GatherScatterMode.PROMISE_IN_BOUNDS)


NACT = 27
WR2 = KS * KS * I_CH
NROWPW = WR2 // NACT
NVEC = O_CH // 16


def _sc_vq_body(w_hbm, m_hbm, d_hbm, q_hbm, red_hbm, w_v, q_v, m_v, d_v,
                red_v):
    wid = lax.axis_index("s") * 2 + lax.axis_index("c")
    base = wid * NROWPW
    pltpu.sync_copy(m_hbm, m_v)
    pltpu.sync_copy(d_hbm, d_v)
    lane = jax.lax.broadcasted_iota(jnp.int32, (16,), 0)
    zero = jnp.zeros((16,), jnp.float32)
    mrows = [m_v[b, :] for b in range(16)]
    mcol = []
    for r in range(16):
        col = zero
        ridx = jnp.full((16,), r, jnp.int32)
        for b in range(16):
            col = jnp.where(lane == b, _vperm(mrows[b], ridx), col)
        mcol.append(col)
    drows = [d_v[b, :] for b in range(16)]
    dcol = {}
    for r in range(0, 16, 2):
        col = zero
        ridx = jnp.full((16,), r, jnp.int32)
        for b in range(16):
            col = jnp.where(lane == b, _vperm(drows[b], ridx), col)
        dcol[r] = col
    m127 = _vperm(mrows[7], jnp.full((16,), 15, jnp.int32))

    def search(w):
        pos = jnp.where(m127 < w, 128, 0)
        gm = _vperm(mcol[15], (pos >> 4) + 3)
        pos = jnp.where(gm < w, pos + 64, pos)
        gm = _vperm(mcol[15], (pos >> 4) + 1)
        pos = jnp.where(gm < w, pos + 32, pos)
        gm = _vperm(mcol[15], pos >> 4)
        pos = jnp.where(gm < w, pos + 16, pos)
        row = pos >> 4
        gm = _vperm(mcol[7], row)
        pos = jnp.where(gm < w, pos + 8, pos)
        gm = jnp.where((pos & 8) != 0, _vperm(mcol[11], row),
                       _vperm(mcol[3], row))
        pos = jnp.where(gm < w, pos + 4, pos)
        ga = jnp.where((pos & 4) != 0, _vperm(mcol[5], row),
                       _vperm(mcol[1], row))
        gb = jnp.where((pos & 4) != 0, _vperm(mcol[13], row),
                       _vperm(mcol[9], row))
        gm = jnp.where((pos & 8) != 0, gb, ga)
        pos = jnp.where(gm < w, pos + 2, pos)
        b2 = (pos & 2) != 0
        b4 = (pos & 4) != 0
        b8 = (pos & 8) != 0
        t0 = jnp.where(b2, _vperm(mcol[2], row), _vperm(mcol[0], row))
        t1 = jnp.where(b2, _vperm(mcol[6], row), _vperm(mcol[4], row))
        t2 = jnp.where(b2, _vperm(mcol[10], row), _vperm(mcol[8], row))
        t3 = jnp.where(b2, _vperm(mcol[14], row), _vperm(mcol[12], row))
        gm = jnp.where(b8, jnp.where(b4, t3, t2), jnp.where(b4, t1, t0))
        d0 = jnp.where(b2, _vperm(dcol[2], row), _vperm(dcol[0], row))
        d1 = jnp.where(b2, _vperm(dcol[6], row), _vperm(dcol[4], row))
        d2 = jnp.where(b2, _vperm(dcol[10], row), _vperm(dcol[8], row))
        d3 = jnp.where(b2, _vperm(dcol[14], row), _vperm(dcol[12], row))
        dv = jnp.where(b8, jnp.where(b4, d3, d2), jnp.where(b4, d1, d0))
        return gm + jnp.where(gm < w, 0.5, -0.5) * dv

    red_v[0, :] = zero
    red_v[1, :] = zero
    red_v[2, :] = zero
    init = (zero,) * (3 * SUBV)

    def trip(r, acc):
        accs = list(acc)
        for j in range(NVEC):
            w = w_v[r, pl.ds(j * 16, 16)]
            q = search(w)
            q_v[r, pl.ds(j * 16, 16)] = q
            e = q - w
            k = 3 * (j % SUBV)
            accs[k] = accs[k] + e * e
            accs[k + 1] = accs[k + 1] + (w * w + q * q - 2.0 * w * q)
            accs[k + 2] = jnp.maximum(accs[k + 2], jnp.abs(q))
        return tuple(accs)

    @pl.when(wid < NACT)
    def _():
        pltpu.sync_copy(w_hbm.at[pl.ds(base, NROWPW), :], w_v)
        acc = lax.fori_loop(0, NROWPW, trip, init)
        red_v[0, :] = acc[0] + acc[3]
        red_v[1, :] = acc[1] + acc[4]
        red_v[2, :] = jnp.maximum(acc[2], acc[5])
        pltpu.sync_copy(q_v, q_hbm.at[pl.ds(base, NROWPW), :])

    pltpu.sync_copy(red_v, red_hbm.at[wid])


def _conv_body(x_ref, w9_ref, red_ref, out_ref, loss_ref):
    b = pl.program_id(0)
    red = red_ref[...]
    esum = jnp.sum(red[:, 0, :])
    dsum = jnp.sum(red[:, 1, :])
    qmax = jnp.max(red[:, 2, :])
    max_abs = jnp.where(qmax > 0.0, qmax, 1.0)

    w9 = w9_ref[...]
    wpos = jnp.maximum(w9, 0.0)
    wneg = jnp.maximum(-w9, 0.0)
    wint = jnp.round(wpos / max_abs * MAXV) - jnp.round(wneg / max_abs * MAXV)
    wb = wint.astype(jnp.bfloat16)

    x = x_ref[0]
    xq = jnp.round(jnp.clip(x, -8.0, 8.0 - 0.0625) * 16.0)
    xb = xq.astype(jnp.bfloat16)

    acc = jnp.zeros((OS * OS, O_CH), jnp.float32)
    for t in range(KS * KS):
        dy, dx = t // KS, t % KS
        p = jax.lax.dot(xb, wb[t], preferred_element_type=jnp.float32)
        pw = p.reshape(SP, SP, O_CH)[dy:dy + OS, dx:dx + OS, :]
        acc = acc + pw.reshape(OS * OS, O_CH)
    out_ref[0] = acc * (max_abs / (MAXV * 16.0))

    @pl.when(b == 0)
    def _():
        e_l = esum / NW
        avg = dsum / NW
        scale = jnp.where(avg < 0.001, 0.1, jnp.where(avg < 0.01, 0.5, 1.0))
        loss = e_l + COMMIT * scale * e_l
        loss_ref[...] = jnp.full((1, 128), loss)


def kernel(x, weight, codebook):
    cb_row = codebook.reshape(1, NEMB)
    cb_col = codebook.reshape(NEMB, 1)
    s, m, d = pl.pallas_call(
        _sort_body,
        out_shape=[jax.ShapeDtypeStruct((1, NEMB), jnp.float32)] * 3,
    )(cb_row, cb_col)

    sc_vq = functools.partial(
        pl.kernel,
        out_type=[jax.ShapeDtypeStruct((WR2, O_CH), jnp.float32),
                  jax.ShapeDtypeStruct((NWORK, 3, 16), jnp.float32)],
        mesh=plsc.VectorSubcoreMesh(core_axis_name="c", subcore_axis_name="s"),
        scratch_types=[pltpu.VMEM((NROWPW, O_CH), jnp.float32),
                       pltpu.VMEM((NROWPW, O_CH), jnp.float32),
                       pltpu.VMEM((16, 16), jnp.float32),
                       pltpu.VMEM((16, 16), jnp.float32),
                       pltpu.VMEM((3, 16), jnp.float32)],
    )(_sc_vq_body)
    w_tco = weight.reshape(O_CH, I_CH, KS * KS).transpose(2, 1, 0)
    q_flat, red = sc_vq(w_tco.reshape(WR2, O_CH), m.reshape(16, 16),
                        d.reshape(16, 16))

    w9 = q_flat.reshape(KS * KS, I_CH, O_CH)
    xpad = jnp.pad(x, ((0, 0), (0, 0), (1, 1), (1, 1)))
    xt = xpad.transpose(0, 2, 3, 1).reshape(B, SP * SP, I_CH)

    out_t, loss_arr = pl.pallas_call(
        _conv_body,
        grid=(B,),
        in_specs=[pl.BlockSpec((1, SP * SP, I_CH), lambda b: (b, 0, 0)),
                  pl.BlockSpec((KS * KS, I_CH, O_CH), lambda b: (0, 0, 0)),
                  pl.BlockSpec((NWORK, 3, 16), lambda b: (0, 0, 0))],
        out_specs=[pl.BlockSpec((1, OS * OS, O_CH), lambda b: (b, 0, 0)),
                   pl.BlockSpec((1, 128), lambda b: (0, 0))],
        out_shape=[jax.ShapeDtypeStruct((B, OS * OS, O_CH), jnp.float32),
                   jax.ShapeDtypeStruct((1, 128), jnp.float32)],
    )(xt, w9, red)

    out = out_t.transpose(0, 2, 1).reshape(B, O_CH, OS, OS)
    return out, loss_arr[0, 0]
